# baseline (speedup 1.0000x reference)
import functools

import jax
import jax.numpy as jnp
from jax import lax
from jax.experimental import pallas as pl
from jax.experimental.pallas import tpu as pltpu
from jax.experimental.pallas import tpu_sc as plsc

B, S = 16, 1500
CODES, SW, W = 1024, 768, 1024
NT = B * S
SC = 3
C = SC * B
NCHUNKS = NT // C
NWORKERS = 32
LANES = 16
KMAX = (NCHUNKS + NWORKERS - 1) // NWORKERS
NFULL = NCHUNKS - (KMAX - 1) * NWORKERS


def _mm_body(a_ref, b_ref, bias_ref, o_ref):
    o_ref[...] = (
        jnp.dot(a_ref[...], b_ref[...], preferred_element_type=jnp.float32,
                precision=lax.Precision.HIGHEST)
        + bias_ref[...]
    )


def _project_table(main_w, e2h_w, e2h_b):
    return pl.pallas_call(
        _mm_body,
        out_shape=jax.ShapeDtypeStruct((CODES, W), jnp.float32),
    )(main_w, e2h_w, e2h_b.reshape(1, W))


def _sc_body(table, idxs, oidx, pos_emb, out,
             idx_v, oidx_v, pos_v, rows_v, sem_g, sem_s):
    wid = lax.axis_index("s") * 2 + lax.axis_index("c")
    is_full = wid < NFULL

    def issue(k, p):
        c = wid + NWORKERS * k
        base = c * C
        pltpu.sync_copy(idxs.at[pl.ds(base, C)], idx_v[p])
        pltpu.sync_copy(oidx.at[pl.ds(base, C)], oidx_v[p])
        pltpu.sync_copy(pos_emb.at[pl.ds(c * SC, SC)], pos_v[p])
        pltpu.async_copy(table.at[idx_v[p]], rows_v[p], sem_g[p])

    def wait_gather(p):
        pltpu.make_async_copy(table.at[idx_v[p]], rows_v[p], sem_g[p]).wait()

    def wait_scatter(p):
        pltpu.make_async_copy(rows_v[p], out.at[oidx_v[p]], sem_s[p]).wait()

    def add_pos(p):
        def row_add(i, carry):
            sl = i // B
            for j in range(W // LANES):
                plsc.addupdate(rows_v[p].at[i, pl.ds(LANES * j, LANES)],
                               pos_v[p][sl, pl.ds(LANES * j, LANES)])
            return carry
        lax.fori_loop(0, C, row_add, 0, unroll=False)

    issue(0, 0)
    for k in range(KMAX):
        p = k % 2
        q = 1 - p
        guard_k = is_full if k == KMAX - 1 else None
        if k + 1 < KMAX:
            if k >= 1:
                wait_scatter(q)
            if k + 1 == KMAX - 1:
                @pl.when(is_full)
                def _():
                    issue(k + 1, q)
            else:
                issue(k + 1, q)
        if guard_k is None:
            wait_gather(p)
            add_pos(p)
            pltpu.async_copy(rows_v[p], out.at[oidx_v[p]], sem_s[p])
        else:
            @pl.when(guard_k)
            def _():
                wait_gather(p)
                add_pos(p)
                pltpu.async_copy(rows_v[p], out.at[oidx_v[p]], sem_s[p])
    wait_scatter((KMAX - 2) % 2)

    @pl.when(is_full)
    def _():
        wait_scatter((KMAX - 1) % 2)


@functools.partial(
    pl.kernel,
    out_type=jax.ShapeDtypeStruct((NT, W), jnp.float32),
    mesh=plsc.VectorSubcoreMesh(core_axis_name="c", subcore_axis_name="s"),
    scratch_types=[
        [pltpu.VMEM((C,), jnp.int32)] * 2,
        [pltpu.VMEM((C,), jnp.int32)] * 2,
        [pltpu.VMEM((SC, W), jnp.float32)] * 2,
        [pltpu.VMEM((C, W), jnp.float32)] * 2,
        [pltpu.SemaphoreType.DMA] * 2,
        [pltpu.SemaphoreType.DMA] * 2,
    ],
    compiler_params=pltpu.CompilerParams(use_tc_tiling_on_sc=False),
)
def _sc_gather_add(table, idxs, oidx, pos_emb, out,
                   idx_v, oidx_v, pos_v, rows_v, sem_g, sem_s):
    _sc_body(table, idxs, oidx, pos_emb, out,
             idx_v, oidx_v, pos_v, rows_v, sem_g, sem_s)


def kernel(Stoks, xenc, main_w, special_w, e2h_w, e2h_b, pos_emb):
    proj = _project_table(main_w, e2h_w, e2h_b)
    table = jnp.concatenate([proj, special_w], axis=0)
    idxs = jnp.transpose(Stoks).reshape(NT).astype(jnp.int32)
    t = jnp.arange(NT, dtype=jnp.int32)
    oidx = (t % B) * S + t // B
    out = _sc_gather_add(table, idxs, oidx, pos_emb)
    return (out.reshape(B, S, W).astype(xenc.dtype), 0)
